# Initial kernel scaffold; baseline (speedup 1.0000x reference)
import jax
import jax.numpy as jnp
from jax.experimental import pallas as pl


def kernel(x_DL, codebook_KL, training):
    raise NotImplementedError("write your pallas kernel here")


# speedup vs baseline: 1.0588x; 1.0588x over previous
import functools

import jax
import jax.numpy as jnp
from jax import lax
from jax.experimental import pallas as pl
from jax.experimental.pallas import tpu as pltpu
from jax.experimental.pallas import tpu_sc as plsc


def _argmax_body(x_ref, cb_ref, xn_ref, idx_ref):
    K = cb_ref.shape[0]
    x = x_ref[...]
    xn = x / (jnp.sqrt(jnp.sum(x * x, axis=1, keepdims=True)) + 1e-8)
    xn_ref[...] = xn

    cb = cb_ref[...]
    cbn = cb / (jnp.sqrt(jnp.sum(cb * cb, axis=1, keepdims=True)) + 1e-8)

    s = lax.dot_general(xn, cbn, (((1,), (1,)), ((), ())),
                        preferred_element_type=jnp.float32)
    m = jnp.max(s, axis=1, keepdims=True)
    ii = lax.broadcasted_iota(jnp.int32, s.shape, 1)
    idx = jnp.min(jnp.where(s == m, ii, K), axis=1)
    idx_ref[...] = idx


def _tc_argmax(x_DL, codebook_KL, block_d):
    D, L = x_DL.shape
    K = codebook_KL.shape[0]
    grid = (D // block_d,)
    return pl.pallas_call(
        _argmax_body,
        grid=grid,
        in_specs=[
            pl.BlockSpec((block_d, L), lambda i: (i, 0)),
            pl.BlockSpec((K, L), lambda i: (0, 0)),
        ],
        out_specs=[
            pl.BlockSpec((block_d, L), lambda i: (i, 0)),
            pl.BlockSpec((block_d,), lambda i: (i,)),
        ],
        out_shape=[
            jax.ShapeDtypeStruct((D, L), jnp.float32),
            jax.ShapeDtypeStruct((D,), jnp.int32),
        ],
    )(x_DL, codebook_KL)


_SC_CHUNK = 128


def _sc_gather(codebook_KL, indices_D):
    D = indices_D.shape[0]
    L = codebook_KL.shape[1]
    info = plsc.get_sparse_core_info()
    nw = info.num_cores * info.num_subcores
    b_per_w = D // nw
    n_chunks = b_per_w // _SC_CHUNK
    mesh = plsc.VectorSubcoreMesh(core_axis_name="c", subcore_axis_name="s")

    @functools.partial(
        pl.kernel,
        mesh=mesh,
        compiler_params=pltpu.CompilerParams(use_tc_tiling_on_sc=False),
        out_type=jax.ShapeDtypeStruct((D, L), jnp.float32),
        scratch_types=[
            pltpu.VMEM((b_per_w,), jnp.int32),
            pltpu.VMEM((b_per_w, L), jnp.float32),
            pltpu.SemaphoreType.DMA,
        ],
    )
    def gather_kernel(table_hbm, idx_hbm, out_hbm, idx_v, rows_v, sem):
        wid = lax.axis_index("s") * info.num_cores + lax.axis_index("c")
        base = wid * b_per_w
        pltpu.sync_copy(idx_hbm.at[pl.ds(base, b_per_w)], idx_v)
        for j in range(n_chunks):
            pltpu.async_copy(
                table_hbm.at[idx_v.at[pl.ds(j * _SC_CHUNK, _SC_CHUNK)]],
                rows_v.at[pl.ds(j * _SC_CHUNK, _SC_CHUNK)],
                sem,
            ).wait()
        pltpu.sync_copy(rows_v, out_hbm.at[pl.ds(base, b_per_w)])

    return gather_kernel(codebook_KL, indices_D)


def kernel(x_DL, codebook_KL, training):
    xn_DL, indices_D = _tc_argmax(x_DL, codebook_KL, block_d=256)
    z_DL = _sc_gather(codebook_KL, indices_D)
    return (z_DL, z_DL, xn_DL, indices_D)


# speedup vs baseline: 1.5501x; 1.4641x over previous
import functools

import jax
import jax.numpy as jnp
from jax import lax
from jax.experimental import pallas as pl
from jax.experimental.pallas import tpu as pltpu
from jax.experimental.pallas import tpu_sc as plsc


def _argmax_body(x_ref, cbT_ref, xn_ref, idx_ref, bs_ref):
    K = cbT_ref.shape[1]

    @pl.when(pl.program_id(0) == 0)
    def _prep():
        c = cbT_ref[...]
        cn = c / (jnp.sqrt(jnp.sum(c * c, axis=0, keepdims=True)) + 1e-8)
        bs_ref[...] = cn.astype(jnp.bfloat16)

    x = x_ref[...]
    xn = x / (jnp.sqrt(jnp.sum(x * x, axis=1, keepdims=True)) + 1e-8)
    xn_ref[...] = xn

    s = lax.dot_general(xn.astype(jnp.bfloat16), bs_ref[...],
                        (((1,), (0,)), ((), ())),
                        preferred_element_type=jnp.float32)
    m = jnp.max(s, axis=1, keepdims=True)
    ii = lax.broadcasted_iota(jnp.int32, s.shape, 1)
    idx = jnp.min(jnp.where(s == m, ii, K), axis=1)
    idx_ref[...] = idx


def _tc_argmax(x_DL, codebook_KL, block_d):
    D, L = x_DL.shape
    K = codebook_KL.shape[0]
    grid = (D // block_d,)
    return pl.pallas_call(
        _argmax_body,
        grid=grid,
        in_specs=[
            pl.BlockSpec((block_d, L), lambda i: (i, 0)),
            pl.BlockSpec((L, K), lambda i: (0, 0)),
        ],
        out_specs=[
            pl.BlockSpec((block_d, L), lambda i: (i, 0)),
            pl.BlockSpec((block_d,), lambda i: (i,)),
        ],
        out_shape=[
            jax.ShapeDtypeStruct((D, L), jnp.float32),
            jax.ShapeDtypeStruct((D,), jnp.int32),
        ],
        scratch_shapes=[pltpu.VMEM((L, K), jnp.bfloat16)],
    )(x_DL, codebook_KL.T)


_SC_CHUNK = 128


def _sc_gather(codebook_KL, indices_D):
    D = indices_D.shape[0]
    L = codebook_KL.shape[1]
    info = plsc.get_sparse_core_info()
    nw = info.num_cores * info.num_subcores
    b_per_w = D // nw
    n_chunks = b_per_w // _SC_CHUNK
    mesh = plsc.VectorSubcoreMesh(core_axis_name="c", subcore_axis_name="s")

    @functools.partial(
        pl.kernel,
        mesh=mesh,
        compiler_params=pltpu.CompilerParams(use_tc_tiling_on_sc=False),
        out_type=jax.ShapeDtypeStruct((D, L), jnp.float32),
        scratch_types=[
            pltpu.VMEM((b_per_w,), jnp.int32),
            pltpu.VMEM((b_per_w, L), jnp.float32),
            pltpu.SemaphoreType.DMA,
        ],
    )
    def gather_kernel(table_hbm, idx_hbm, out_hbm, idx_v, rows_v, sem):
        wid = lax.axis_index("s") * info.num_cores + lax.axis_index("c")
        base = wid * b_per_w
        pltpu.sync_copy(idx_hbm.at[pl.ds(base, b_per_w)], idx_v)
        for j in range(n_chunks):
            pltpu.async_copy(
                table_hbm.at[idx_v.at[pl.ds(j * _SC_CHUNK, _SC_CHUNK)]],
                rows_v.at[pl.ds(j * _SC_CHUNK, _SC_CHUNK)],
                sem,
            ).wait()
        pltpu.sync_copy(rows_v, out_hbm.at[pl.ds(base, b_per_w)])

    return gather_kernel(codebook_KL, indices_D)


def kernel(x_DL, codebook_KL, training):
    xn_DL, indices_D = _tc_argmax(x_DL, codebook_KL, block_d=512)
    z_DL = _sc_gather(codebook_KL, indices_D)
    return (z_DL, z_DL, xn_DL, indices_D)


# speedup vs baseline: 1.7799x; 1.1483x over previous
import functools

import jax
import jax.numpy as jnp
from jax import lax
from jax.experimental import pallas as pl
from jax.experimental.pallas import tpu as pltpu
from jax.experimental.pallas import tpu_sc as plsc


def _argmax_body(x_ref, cbT_ref, xn_ref, idx_ref, bs_ref, io_ref):
    K = cbT_ref.shape[1]

    @pl.when(pl.program_id(0) == 0)
    def _prep():
        c = cbT_ref[...]
        cn = c / (jnp.sqrt(jnp.sum(c * c, axis=0, keepdims=True)) + 1e-8)
        bs_ref[...] = cn.astype(jnp.bfloat16)
        io_ref[...] = lax.broadcasted_iota(jnp.int32, (1, K),
                                           1).astype(jnp.float32)

    x = x_ref[...]
    xn = x / (jnp.sqrt(jnp.sum(x * x, axis=1, keepdims=True)) + 1e-8)
    xn_ref[...] = xn

    s = lax.dot_general(xn.astype(jnp.bfloat16), bs_ref[...],
                        (((1,), (0,)), ((), ())),
                        preferred_element_type=jnp.float32)
    m = jnp.max(s, axis=1, keepdims=True)
    idxf = jnp.min(jnp.where(s == m, io_ref[...], float(K)), axis=1)
    idx_ref[...] = idxf.astype(jnp.int32)


def _tc_argmax(x_DL, codebook_KL, block_d):
    D, L = x_DL.shape
    K = codebook_KL.shape[0]
    grid = (D // block_d,)
    return pl.pallas_call(
        _argmax_body,
        grid=grid,
        in_specs=[
            pl.BlockSpec((block_d, L), lambda i: (i, 0)),
            pl.BlockSpec((L, K), lambda i: (0, 0)),
        ],
        out_specs=[
            pl.BlockSpec((block_d, L), lambda i: (i, 0)),
            pl.BlockSpec((block_d,), lambda i: (i,)),
        ],
        out_shape=[
            jax.ShapeDtypeStruct((D, L), jnp.float32),
            jax.ShapeDtypeStruct((D,), jnp.int32),
        ],
        scratch_shapes=[pltpu.VMEM((L, K), jnp.bfloat16),
                        pltpu.VMEM((1, K), jnp.float32)],
    )(x_DL, codebook_KL.T)


_SC_CHUNK = 128


def _sc_gather(codebook_KL, indices_D):
    D = indices_D.shape[0]
    L = codebook_KL.shape[1]
    info = plsc.get_sparse_core_info()
    nw = info.num_cores * info.num_subcores
    b_per_w = D // nw
    n_chunks = b_per_w // _SC_CHUNK
    mesh = plsc.VectorSubcoreMesh(core_axis_name="c", subcore_axis_name="s")

    @functools.partial(
        pl.kernel,
        mesh=mesh,
        compiler_params=pltpu.CompilerParams(use_tc_tiling_on_sc=False),
        out_type=jax.ShapeDtypeStruct((D, L), jnp.float32),
        scratch_types=[
            pltpu.VMEM((b_per_w,), jnp.int32),
            pltpu.VMEM((b_per_w, L), jnp.float32),
            pltpu.SemaphoreType.DMA,
        ],
    )
    def gather_kernel(table_hbm, idx_hbm, out_hbm, idx_v, rows_v, sem):
        wid = lax.axis_index("s") * info.num_cores + lax.axis_index("c")
        base = wid * b_per_w
        pltpu.sync_copy(idx_hbm.at[pl.ds(base, b_per_w)], idx_v)
        for j in range(n_chunks):
            pltpu.async_copy(
                table_hbm.at[idx_v.at[pl.ds(j * _SC_CHUNK, _SC_CHUNK)]],
                rows_v.at[pl.ds(j * _SC_CHUNK, _SC_CHUNK)],
                sem,
            ).wait()
        pltpu.sync_copy(rows_v, out_hbm.at[pl.ds(base, b_per_w)])

    return gather_kernel(codebook_KL, indices_D)


def kernel(x_DL, codebook_KL, training):
    xn_DL, indices_D = _tc_argmax(x_DL, codebook_KL, block_d=512)
    z_DL = _sc_gather(codebook_KL, indices_D)
    return (z_DL, z_DL, xn_DL, indices_D)
